# Initial kernel scaffold; baseline (speedup 1.0000x reference)
import jax
import jax.numpy as jnp
from jax.experimental import pallas as pl


def kernel(x, edge_index, W1, b1, ln_w, ln_b, prelu_a, W2, b2):
    raise NotImplementedError("write your pallas kernel here")


# speedup vs baseline: 8.8003x; 8.8003x over previous
import functools

import jax
import jax.numpy as jnp
from jax import lax
from jax.experimental import pallas as pl
from jax.experimental.pallas import tpu as pltpu
from jax.experimental.pallas import tpu_sc as plsc

N = 10000
E = 320000
D = 128

NC = 2
NS = 16
NW = NC * NS
CHUNK = 128
K = 80
E_PAD = NW * K * CHUNK
EPT = E_PAD // NW
N_PAD = 10112
RPT = N_PAD // NS

BLK = 400
GRID = N // BLK

@functools.cache
def _build_hist():
    mesh = plsc.VectorSubcoreMesh(core_axis_name="c", subcore_axis_name="s")
    return functools.partial(
        pl.kernel,
        mesh=mesh,
        out_type=jax.ShapeDtypeStruct((NW, 1, N_PAD), jnp.float32),
        scratch_types=[
            pltpu.VMEM((EPT,), jnp.int32),
            pltpu.VMEM((N_PAD,), jnp.float32),
        ],
        compiler_params=pltpu.CompilerParams(needs_layout_passes=False),
    )(_hist_body)


def _hist_body(dst_hbm, out_hbm, dstv, hist):
    wid = lax.axis_index("c") * NS + lax.axis_index("s")
    pltpu.sync_copy(dst_hbm.at[pl.ds(wid * EPT, EPT)], dstv)

    def _zero(i, carry):
        hist[pl.ds(i * 16, 16)] = jnp.zeros((16,), jnp.float32)
        return carry

    lax.fori_loop(0, N_PAD // 16, _zero, 0)

    ones = jnp.full((16,), 1.0, jnp.float32)

    def _count(i, carry):
        idx = dstv[pl.ds(i * 16, 16)]
        plsc.addupdate_scatter(hist, [idx], ones)
        return carry

    lax.fori_loop(0, EPT // 16, _count, 0)
    pltpu.sync_copy(hist, out_hbm.at[wid, 0])


@functools.cache
def _build_scatter():
    mesh = plsc.VectorSubcoreMesh(core_axis_name="c", subcore_axis_name="s")
    return functools.partial(
        pl.kernel,
        mesh=mesh,
        out_type=jax.ShapeDtypeStruct((NC, N_PAD, D), jnp.float32),
        scratch_types=[
            pltpu.VMEM((K, CHUNK), jnp.int32),
            pltpu.VMEM((K, CHUNK), jnp.int32),
            pltpu.VMEM((CHUNK, D), jnp.float32),
            pltpu.VMEM_SHARED((N_PAD, D), jnp.float32),
            pltpu.SemaphoreType.DMA,
        ],
    )(_scatter_body)


def _scatter_body(h_hbm, src_hbm, dst_hbm, zeros_hbm, out_hbm,
                  srcv, dstv, rows, acc, sem):
    c = lax.axis_index("c")
    s = lax.axis_index("s")
    rbase = (c * NS + s) * K
    pltpu.sync_copy(src_hbm.at[pl.ds(rbase, K)], srcv)
    pltpu.sync_copy(dst_hbm.at[pl.ds(rbase, K)], dstv)
    pltpu.sync_copy(zeros_hbm.at[pl.ds(s * RPT, RPT)],
                    acc.at[pl.ds(s * RPT, RPT)])
    plsc.subcore_barrier()

    def _edge_chunk(j, carry):
        pltpu.async_copy(h_hbm.at[srcv.at[j]], rows, sem).wait()
        pltpu.sync_copy(rows, acc.at[dstv.at[j]], add=True)
        return carry

    lax.fori_loop(0, K, _edge_chunk, 0)
    plsc.subcore_barrier()
    pltpu.sync_copy(acc.at[pl.ds(s * RPT, RPT)],
                    out_hbm.at[c, pl.ds(s * RPT, RPT)])


def _dinv_from_partials(pd):
    deg = jnp.sum(pd, axis=1) + 1.0
    return lax.rsqrt(deg)


def _tc1_body(x_ref, w1t_ref, pd_ref, out_ref):
    dinv = _dinv_from_partials(pd_ref[...])
    h = jnp.dot(x_ref[...], w1t_ref[...], preferred_element_type=jnp.float32)
    out_ref[...] = h * dinv[:, None]


def _tc2_body(acc_ref, h1_ref, pd_ref, b1_ref, lnw_ref, lnb_ref, a_ref,
              w2t_ref, out_ref):
    dinv = _dinv_from_partials(pd_ref[...])
    tot = acc_ref[0] + acc_ref[1] + h1_ref[...]
    h1 = dinv[:, None] * tot + b1_ref[...]
    mu = jnp.mean(h1, axis=-1, keepdims=True)
    xc = h1 - mu
    var = jnp.mean(xc * xc, axis=-1, keepdims=True)
    g = xc * lax.rsqrt(var + 1e-5) * lnw_ref[...] + lnb_ref[...]
    g = jnp.where(g >= 0.0, g, a_ref[0, 0] * g)
    h2 = jnp.dot(g, w2t_ref[...], preferred_element_type=jnp.float32)
    out_ref[...] = h2 * dinv[:, None]


def _tc3_body(acc_ref, h2_ref, pd_ref, b2_ref, out_ref):
    dinv = _dinv_from_partials(pd_ref[...])
    tot = acc_ref[0] + acc_ref[1] + h2_ref[...]
    out_ref[...] = dinv[:, None] * tot + b2_ref[...]


def _row_spec():
    return pl.BlockSpec((BLK, D), lambda i: (i, 0))


def _pd_spec():
    return pl.BlockSpec((BLK, NW), lambda i: (i, 0))


def _full_spec(shape):
    nd = len(shape)
    return pl.BlockSpec(shape, lambda i: (0,) * nd)


def _tc1(x, w1t, pd):
    return pl.pallas_call(
        _tc1_body,
        grid=(GRID,),
        in_specs=[_row_spec(), _full_spec((D, D)), _pd_spec()],
        out_specs=_row_spec(),
        out_shape=jax.ShapeDtypeStruct((N_PAD, D), jnp.float32),
    )(x, w1t, pd)


def _tc2(acc, h1, pd, b1, lnw, lnb, a, w2t):
    return pl.pallas_call(
        _tc2_body,
        grid=(GRID,),
        in_specs=[
            pl.BlockSpec((NC, BLK, D), lambda i: (0, i, 0)),
            _row_spec(), _pd_spec(),
            _full_spec((1, D)), _full_spec((1, D)), _full_spec((1, D)),
            _full_spec((1, 1)), _full_spec((D, D)),
        ],
        out_specs=_row_spec(),
        out_shape=jax.ShapeDtypeStruct((N_PAD, D), jnp.float32),
    )(acc, h1, pd, b1, lnw, lnb, a, w2t)


def _tc3(acc, h2, pd, b2):
    return pl.pallas_call(
        _tc3_body,
        grid=(GRID,),
        in_specs=[
            pl.BlockSpec((NC, BLK, D), lambda i: (0, i, 0)),
            _row_spec(), _pd_spec(), _full_spec((1, D)),
        ],
        out_specs=_row_spec(),
        out_shape=jax.ShapeDtypeStruct((N, D), jnp.float32),
    )(acc, h2, pd, b2)


def kernel(x, edge_index, W1, b1, ln_w, ln_b, prelu_a, W2, b2):
    src = edge_index[0]
    dst = edge_index[1]
    pad = jnp.full((E_PAD - E,), N, jnp.int32)
    src2 = jnp.concatenate([src, pad]).reshape(NW * K, CHUNK)
    dst_p = jnp.concatenate([dst, pad])
    dst2 = dst_p.reshape(NW * K, CHUNK)

    partials = _build_hist()(dst_p)
    partials = partials.reshape(NW, N_PAD).T

    zeros = jnp.zeros((N_PAD, D), jnp.float32)
    b1r = b1.reshape(1, D)
    b2r = b2.reshape(1, D)
    lnwr = ln_w.reshape(1, D)
    lnbr = ln_b.reshape(1, D)
    ar = prelu_a.reshape(1, 1)

    scatter = _build_scatter()
    h1 = _tc1(x, W1.T, partials)
    acc1 = scatter(h1, src2, dst2, zeros)
    h2 = _tc2(acc1, h1, partials, b1r, lnwr, lnbr, ar, W2.T)
    acc2 = scatter(h2, src2, dst2, zeros)
    out = _tc3(acc2, h2, partials, b2r)
    return out


# speedup vs baseline: 14.7642x; 1.6777x over previous
import functools

import jax
import jax.numpy as jnp
from jax import lax
from jax.experimental import pallas as pl
from jax.experimental.pallas import tpu as pltpu
from jax.experimental.pallas import tpu_sc as plsc

N = 10000
E = 320000
D = 128
DH = D // 2

NC = 2
NS = 16
NW = NC * NS
CHUNK = 128
K = 160
E_PAD = NS * K * CHUNK
EPT = E_PAD // NW
N_PAD = 10112
RPT = N_PAD // NS
SETLEN = 2

BLK = 400
GRID = N // BLK

@functools.cache
def _build_hist():
    mesh = plsc.VectorSubcoreMesh(core_axis_name="c", subcore_axis_name="s")
    return functools.partial(
        pl.kernel,
        mesh=mesh,
        out_type=jax.ShapeDtypeStruct((NW, 1, N_PAD), jnp.float32),
        scratch_types=[
            pltpu.VMEM((EPT,), jnp.int32),
            pltpu.VMEM((N_PAD,), jnp.float32),
        ],
        compiler_params=pltpu.CompilerParams(needs_layout_passes=False),
    )(_hist_body)


def _hist_body(dst_hbm, out_hbm, dstv, hist):
    wid = lax.axis_index("c") * NS + lax.axis_index("s")
    pltpu.sync_copy(dst_hbm.at[pl.ds(wid * EPT, EPT)], dstv)

    def _zero(i, carry):
        hist[pl.ds(i * 16, 16)] = jnp.zeros((16,), jnp.float32)
        return carry

    lax.fori_loop(0, N_PAD // 16, _zero, 0)

    ones = jnp.full((16,), 1.0, jnp.float32)

    def _count(i, carry):
        idx = dstv[pl.ds(i * 16, 16)]
        plsc.addupdate_scatter(hist, [idx], ones)
        return carry

    lax.fori_loop(0, EPT // 16, _count, 0)
    pltpu.sync_copy(hist, out_hbm.at[wid, 0])


@functools.cache
def _build_scatter():
    mesh = plsc.VectorSubcoreMesh(core_axis_name="c", subcore_axis_name="s")
    return functools.partial(
        pl.kernel,
        mesh=mesh,
        out_type=jax.ShapeDtypeStruct((NC, N_PAD, DH), jnp.float32),
        scratch_types=[
            pltpu.VMEM((K, CHUNK), jnp.int32),
            pltpu.VMEM((K, CHUNK), jnp.int32),
            pltpu.VMEM((2, SETLEN * CHUNK, DH), jnp.float32),
            pltpu.VMEM_SHARED((N_PAD, DH), jnp.float32),
        ] + [pltpu.SemaphoreType.DMA] * 2,
        compiler_params=pltpu.CompilerParams(use_tc_tiling_on_sc=False),
    )(_scatter_body)


def _scatter_body(h_hbm, src_hbm, dst_hbm, zeros_hbm, out_hbm,
                  srcv, dstv, rows, acc, *sems):
    c = lax.axis_index("c")
    s = lax.axis_index("s")
    table = h_hbm.at[c]
    rbase = s * K
    pltpu.sync_copy(src_hbm.at[pl.ds(rbase, K)], srcv)
    pltpu.sync_copy(dst_hbm.at[pl.ds(rbase, K)], dstv)

    def _fire(g, p):
        for i in range(SETLEN):
            j = g * SETLEN + i
            pltpu.async_copy(table.at[srcv.at[j]],
                             rows.at[p, pl.ds(i * CHUNK, CHUNK)], sems[p])

    _fire(0, 0)
    _fire(1, 1)
    pltpu.sync_copy(zeros_hbm.at[pl.ds(s * RPT, RPT)],
                    acc.at[pl.ds(s * RPT, RPT)])
    plsc.subcore_barrier()

    NG = K // SETLEN

    def _step(g, p):
        pltpu.make_async_copy(table.at[pl.ds(0, SETLEN * CHUNK)],
                              rows.at[p], sems[p]).wait()
        for i in range(SETLEN):
            j = g * SETLEN + i
            pltpu.sync_copy(rows.at[p, pl.ds(i * CHUNK, CHUNK)],
                            acc.at[dstv.at[j]], add=True)

        @pl.when(g + 2 < NG)
        def _():
            _fire(g + 2, p)

    def _pair(gg, carry):
        _step(gg * 2, 0)
        _step(gg * 2 + 1, 1)
        return carry

    lax.fori_loop(0, NG // 2, _pair, 0)
    plsc.subcore_barrier()
    pltpu.sync_copy(acc.at[pl.ds(s * RPT, RPT)],
                    out_hbm.at[c, pl.ds(s * RPT, RPT)])


def _dinv_from_partials(pd):
    deg = jnp.sum(pd, axis=1) + 1.0
    return lax.rsqrt(deg)


def _split_store(out_ref, h):
    out_ref[0, :, :] = h[:, :DH]
    out_ref[1, :, :] = h[:, DH:]


def _combine(acc_ref, h_ref):
    lo = acc_ref[0] + h_ref[0]
    hi = acc_ref[1] + h_ref[1]
    return jnp.concatenate([lo, hi], axis=-1)


def _tc1_body(x_ref, w1t_ref, pd_ref, out_ref):
    dinv = _dinv_from_partials(pd_ref[...])
    h = jnp.dot(x_ref[...], w1t_ref[...], preferred_element_type=jnp.float32)
    _split_store(out_ref, h * dinv[:, None])


def _tc2_body(acc_ref, h1_ref, pd_ref, b1_ref, lnw_ref, lnb_ref, a_ref,
              w2t_ref, out_ref):
    dinv = _dinv_from_partials(pd_ref[...])
    h1 = dinv[:, None] * _combine(acc_ref, h1_ref) + b1_ref[...]
    mu = jnp.mean(h1, axis=-1, keepdims=True)
    xc = h1 - mu
    var = jnp.mean(xc * xc, axis=-1, keepdims=True)
    g = xc * lax.rsqrt(var + 1e-5) * lnw_ref[...] + lnb_ref[...]
    g = jnp.where(g >= 0.0, g, a_ref[0, 0] * g)
    h2 = jnp.dot(g, w2t_ref[...], preferred_element_type=jnp.float32)
    _split_store(out_ref, h2 * dinv[:, None])


def _tc3_body(acc_ref, h2_ref, pd_ref, b2_ref, out_ref):
    dinv = _dinv_from_partials(pd_ref[...])
    out_ref[...] = dinv[:, None] * _combine(acc_ref, h2_ref) + b2_ref[...]


def _row_spec():
    return pl.BlockSpec((BLK, D), lambda i: (i, 0))


def _split_spec():
    return pl.BlockSpec((NC, BLK, DH), lambda i: (0, i, 0))


def _pd_spec():
    return pl.BlockSpec((BLK, NW), lambda i: (i, 0))


def _full_spec(shape):
    nd = len(shape)
    return pl.BlockSpec(shape, lambda i: (0,) * nd)


def _tc1(x, w1t, pd):
    return pl.pallas_call(
        _tc1_body,
        grid=(GRID,),
        in_specs=[_row_spec(), _full_spec((D, D)), _pd_spec()],
        out_specs=_split_spec(),
        out_shape=jax.ShapeDtypeStruct((NC, N_PAD, DH), jnp.float32),
    )(x, w1t, pd)


def _tc2(acc, h1, pd, b1, lnw, lnb, a, w2t):
    return pl.pallas_call(
        _tc2_body,
        grid=(GRID,),
        in_specs=[
            _split_spec(), _split_spec(), _pd_spec(),
            _full_spec((1, D)), _full_spec((1, D)), _full_spec((1, D)),
            _full_spec((1, 1)), _full_spec((D, D)),
        ],
        out_specs=_split_spec(),
        out_shape=jax.ShapeDtypeStruct((NC, N_PAD, DH), jnp.float32),
    )(acc, h1, pd, b1, lnw, lnb, a, w2t)


def _tc3(acc, h2, pd, b2):
    return pl.pallas_call(
        _tc3_body,
        grid=(GRID,),
        in_specs=[
            _split_spec(), _split_spec(), _pd_spec(), _full_spec((1, D)),
        ],
        out_specs=_row_spec(),
        out_shape=jax.ShapeDtypeStruct((N, D), jnp.float32),
    )(acc, h2, pd, b2)


def kernel(x, edge_index, W1, b1, ln_w, ln_b, prelu_a, W2, b2):
    src = edge_index[0]
    dst = edge_index[1]
    pad = jnp.full((E_PAD - E,), N, jnp.int32)
    src2 = jnp.concatenate([src, pad]).reshape(NS * K, CHUNK)
    dst_p = jnp.concatenate([dst, pad])
    dst2 = dst_p.reshape(NS * K, CHUNK)

    partials = _build_hist()(dst_p)
    partials = partials.reshape(NW, N_PAD).T

    zeros = jnp.zeros((N_PAD, DH), jnp.float32)
    b1r = b1.reshape(1, D)
    b2r = b2.reshape(1, D)
    lnwr = ln_w.reshape(1, D)
    lnbr = ln_b.reshape(1, D)
    ar = prelu_a.reshape(1, 1)

    scatter = _build_scatter()
    h1 = _tc1(x, W1.T, partials)
    acc1 = scatter(h1, src2, dst2, zeros)
    h2 = _tc2(acc1, h1, partials, b1r, lnwr, lnbr, ar, W2.T)
    acc2 = scatter(h2, src2, dst2, zeros)
    out = _tc3(acc2, h2, partials, b2r)
    return out


# speedup vs baseline: 14.8191x; 1.0037x over previous
import functools

import jax
import jax.numpy as jnp
from jax import lax
from jax.experimental import pallas as pl
from jax.experimental.pallas import tpu as pltpu
from jax.experimental.pallas import tpu_sc as plsc

N = 10000
E = 320000
D = 128
DH = D // 2

NC = 2
NS = 16
NW = NC * NS
CHUNK = 128
K = 160
E_PAD = NS * K * CHUNK
EPT = E_PAD // NW
N_PAD = 10112
RPT = N_PAD // NS
SETLEN = 2

BLK = 400
GRID = N // BLK

@functools.cache
def _build_hist():
    mesh = plsc.VectorSubcoreMesh(core_axis_name="c", subcore_axis_name="s")
    return functools.partial(
        pl.kernel,
        mesh=mesh,
        out_type=jax.ShapeDtypeStruct((NW, 1, N_PAD), jnp.float32),
        scratch_types=[
            pltpu.VMEM((EPT,), jnp.int32),
            pltpu.VMEM((N_PAD,), jnp.float32),
        ],
        compiler_params=pltpu.CompilerParams(needs_layout_passes=False),
    )(_hist_body)


def _hist_body(dst_hbm, out_hbm, dstv, hist):
    wid = lax.axis_index("c") * NS + lax.axis_index("s")
    pltpu.sync_copy(dst_hbm.at[pl.ds(wid * EPT, EPT)], dstv)

    def _zero(i, carry):
        hist[pl.ds(i * 16, 16)] = jnp.zeros((16,), jnp.float32)
        return carry

    lax.fori_loop(0, N_PAD // 16, _zero, 0)

    ones = jnp.full((16,), 1.0, jnp.float32)

    def _count(i, carry):
        idx = dstv[pl.ds(i * 16, 16)]
        plsc.addupdate_scatter(hist, [idx], ones)
        return carry

    lax.fori_loop(0, EPT // 16, _count, 0)
    pltpu.sync_copy(hist, out_hbm.at[wid, 0])


@functools.cache
def _build_scatter():
    mesh = plsc.VectorSubcoreMesh(core_axis_name="c", subcore_axis_name="s")
    return functools.partial(
        pl.kernel,
        mesh=mesh,
        out_type=jax.ShapeDtypeStruct((NC, N_PAD, DH), jnp.float32),
        scratch_types=[
            pltpu.VMEM((K, CHUNK), jnp.int32),
            pltpu.VMEM((K, CHUNK), jnp.int32),
            pltpu.VMEM((2, SETLEN * CHUNK, DH), jnp.float32),
            pltpu.VMEM_SHARED((N_PAD, DH), jnp.float32),
        ] + [pltpu.SemaphoreType.DMA] * 4,
        compiler_params=pltpu.CompilerParams(use_tc_tiling_on_sc=False),
    )(_scatter_body)


def _scatter_body(h_hbm, src_hbm, dst_hbm, zeros_hbm, out_hbm,
                  srcv, dstv, rows, acc, *sems):
    c = lax.axis_index("c")
    s = lax.axis_index("s")
    table = h_hbm.at[c]
    rbase = s * K
    pltpu.sync_copy(src_hbm.at[pl.ds(rbase, K)], srcv)
    pltpu.sync_copy(dst_hbm.at[pl.ds(rbase, K)], dstv)

    def _fire(g, p):
        for i in range(SETLEN):
            j = g * SETLEN + i
            pltpu.async_copy(table.at[srcv.at[j]],
                             rows.at[p, pl.ds(i * CHUNK, CHUNK)], sems[p])

    _fire(0, 0)
    _fire(1, 1)
    pltpu.sync_copy(zeros_hbm.at[pl.ds(s * RPT, RPT)],
                    acc.at[pl.ds(s * RPT, RPT)])
    plsc.subcore_barrier()

    NG = K // SETLEN

    def _step(g, p):
        pltpu.make_async_copy(table.at[pl.ds(0, SETLEN * CHUNK)],
                              rows.at[p], sems[p]).wait()
        for i in range(SETLEN):
            j = g * SETLEN + i
            pltpu.make_async_copy(rows.at[p, pl.ds(i * CHUNK, CHUNK)],
                                  acc.at[dstv.at[j]],
                                  sems[2 + p]).start(add=True)
        for i in range(SETLEN):
            j = g * SETLEN + i
            pltpu.make_async_copy(rows.at[p, pl.ds(i * CHUNK, CHUNK)],
                                  acc.at[dstv.at[j]], sems[2 + p]).wait()

        @pl.when(g + 2 < NG)
        def _():
            _fire(g + 2, p)

    def _pair(gg, carry):
        _step(gg * 2, 0)
        _step(gg * 2 + 1, 1)
        return carry

    lax.fori_loop(0, NG // 2, _pair, 0)
    plsc.subcore_barrier()
    pltpu.sync_copy(acc.at[pl.ds(s * RPT, RPT)],
                    out_hbm.at[c, pl.ds(s * RPT, RPT)])


def _dinv_from_partials(pd):
    deg = jnp.sum(pd, axis=1) + 1.0
    return lax.rsqrt(deg)


def _split_store(out_ref, h):
    out_ref[0, :, :] = h[:, :DH]
    out_ref[1, :, :] = h[:, DH:]


def _combine(acc_ref, h_ref):
    lo = acc_ref[0] + h_ref[0]
    hi = acc_ref[1] + h_ref[1]
    return jnp.concatenate([lo, hi], axis=-1)


def _tc1_body(x_ref, w1t_ref, pd_ref, out_ref):
    dinv = _dinv_from_partials(pd_ref[...])
    h = jnp.dot(x_ref[...], w1t_ref[...], preferred_element_type=jnp.float32)
    _split_store(out_ref, h * dinv[:, None])


def _tc2_body(acc_ref, h1_ref, pd_ref, b1_ref, lnw_ref, lnb_ref, a_ref,
              w2t_ref, out_ref):
    dinv = _dinv_from_partials(pd_ref[...])
    h1 = dinv[:, None] * _combine(acc_ref, h1_ref) + b1_ref[...]
    mu = jnp.mean(h1, axis=-1, keepdims=True)
    xc = h1 - mu
    var = jnp.mean(xc * xc, axis=-1, keepdims=True)
    g = xc * lax.rsqrt(var + 1e-5) * lnw_ref[...] + lnb_ref[...]
    g = jnp.where(g >= 0.0, g, a_ref[0, 0] * g)
    h2 = jnp.dot(g, w2t_ref[...], preferred_element_type=jnp.float32)
    _split_store(out_ref, h2 * dinv[:, None])


def _tc3_body(acc_ref, h2_ref, pd_ref, b2_ref, out_ref):
    dinv = _dinv_from_partials(pd_ref[...])
    out_ref[...] = dinv[:, None] * _combine(acc_ref, h2_ref) + b2_ref[...]


def _row_spec():
    return pl.BlockSpec((BLK, D), lambda i: (i, 0))


def _split_spec():
    return pl.BlockSpec((NC, BLK, DH), lambda i: (0, i, 0))


def _pd_spec():
    return pl.BlockSpec((BLK, NW), lambda i: (i, 0))


def _full_spec(shape):
    nd = len(shape)
    return pl.BlockSpec(shape, lambda i: (0,) * nd)


def _tc1(x, w1t, pd):
    return pl.pallas_call(
        _tc1_body,
        grid=(GRID,),
        in_specs=[_row_spec(), _full_spec((D, D)), _pd_spec()],
        out_specs=_split_spec(),
        out_shape=jax.ShapeDtypeStruct((NC, N_PAD, DH), jnp.float32),
    )(x, w1t, pd)


def _tc2(acc, h1, pd, b1, lnw, lnb, a, w2t):
    return pl.pallas_call(
        _tc2_body,
        grid=(GRID,),
        in_specs=[
            _split_spec(), _split_spec(), _pd_spec(),
            _full_spec((1, D)), _full_spec((1, D)), _full_spec((1, D)),
            _full_spec((1, 1)), _full_spec((D, D)),
        ],
        out_specs=_split_spec(),
        out_shape=jax.ShapeDtypeStruct((NC, N_PAD, DH), jnp.float32),
    )(acc, h1, pd, b1, lnw, lnb, a, w2t)


def _tc3(acc, h2, pd, b2):
    return pl.pallas_call(
        _tc3_body,
        grid=(GRID,),
        in_specs=[
            _split_spec(), _split_spec(), _pd_spec(), _full_spec((1, D)),
        ],
        out_specs=_row_spec(),
        out_shape=jax.ShapeDtypeStruct((N, D), jnp.float32),
    )(acc, h2, pd, b2)


def kernel(x, edge_index, W1, b1, ln_w, ln_b, prelu_a, W2, b2):
    src = edge_index[0]
    dst = edge_index[1]
    pad = jnp.full((E_PAD - E,), N, jnp.int32)
    src2 = jnp.concatenate([src, pad]).reshape(NS * K, CHUNK)
    dst_p = jnp.concatenate([dst, pad])
    dst2 = dst_p.reshape(NS * K, CHUNK)

    partials = _build_hist()(dst_p)
    partials = partials.reshape(NW, N_PAD).T

    zeros = jnp.zeros((N_PAD, DH), jnp.float32)
    b1r = b1.reshape(1, D)
    b2r = b2.reshape(1, D)
    lnwr = ln_w.reshape(1, D)
    lnbr = ln_b.reshape(1, D)
    ar = prelu_a.reshape(1, 1)

    scatter = _build_scatter()
    h1 = _tc1(x, W1.T, partials)
    acc1 = scatter(h1, src2, dst2, zeros)
    h2 = _tc2(acc1, h1, partials, b1r, lnwr, lnbr, ar, W2.T)
    acc2 = scatter(h2, src2, dst2, zeros)
    out = _tc3(acc2, h2, partials, b2r)
    return out
